# Initial kernel scaffold; baseline (speedup 1.0000x reference)
import jax
import jax.numpy as jnp
from jax.experimental import pallas as pl


def kernel(x, W_embed, b_embed, W1, b1, W2, b2, W3, b3):
    raise NotImplementedError("write your pallas kernel here")


# speedup vs baseline: 71.9080x; 71.9080x over previous
import functools

import numpy as np
import jax
import jax.numpy as jnp
from jax.experimental import pallas as pl
from jax.experimental.pallas import tpu as pltpu

_B, _C, _IMG, _P = 32, 3, 224, 16
_HID = 128
_G = _IMG // _P
_NP = _G * _G
_N = _B * _NP
_ND = _C * _P * _P
_APAD = 256
_RBLK = 448
_NSTEP = _N // _RBLK


@functools.lru_cache(maxsize=1)
def _static_graph():
    r, c = np.triu_indices(_NP, k=1)
    e = np.stack([r.astype(np.int64), c.astype(np.int64)])
    e = np.tile(e[None], (_B, 1, 1)).reshape(-1, 2).T
    row, col = e[0], e[1]
    deg = np.zeros((_N,), np.float64)
    np.add.at(deg, col, 1.0)
    deg += 1.0
    dinv = deg ** -0.5
    A = np.zeros((_APAD, _APAD), np.float64)
    np.add.at(A, (col, row), dinv[row] * dinv[col])
    idx = np.arange(_APAD)
    A[idx, idx] += dinv[:_APAD] ** 2
    Sp = np.zeros((_B, _N), np.float64)
    Sp[0, :_APAD] = A[:_NP, :].sum(axis=0) / _NP
    for g in range(1, _B):
        Sp[g, g * _NP:(g + 1) * _NP] = 1.0 / _NP
    return A.astype(np.float32), Sp.astype(np.float32)


def _fused_body(patches_ref, wemb_ref, bemb_ref, w1_ref, b1_ref, w2_ref,
                b2_ref, w3_ref, b3_ref, a_ref, sp_ref, out_ref,
                xh_ref, t_ref):
    i = pl.program_id(0)
    xh_ref[pl.ds(i * _RBLK, _RBLK), :] = (
        jnp.dot(patches_ref[...], wemb_ref[...],
                preferred_element_type=jnp.float32) + bemb_ref[...])

    @pl.when(i == _NSTEP - 1)
    def _tail():
        a = a_ref[...]
        h = xh_ref[...]
        for w_ref, b_ref in ((w1_ref, b1_ref), (w2_ref, b2_ref)):
            t = jnp.dot(h, w_ref[...], preferred_element_type=jnp.float32)
            t_ref[...] = t
            t_ref[0:_APAD, :] = jnp.dot(a, t[0:_APAD, :],
                                        preferred_element_type=jnp.float32)
            h = jnp.maximum(t_ref[...] + b_ref[...], 0.0)
        p = jnp.dot(sp_ref[...], h, preferred_element_type=jnp.float32)
        out_ref[...] = (jnp.dot(p, w3_ref[...],
                                preferred_element_type=jnp.float32)
                        + b3_ref[...])


def kernel(x, W_embed, b_embed, W1, b1, W2, b2, W3, b3):
    A, Sp = _static_graph()
    patches = (x.reshape(_B, _C, _G, _P, _G, _P)
               .transpose(0, 2, 4, 1, 3, 5)
               .reshape(_N, _ND))
    full = lambda shape: pl.BlockSpec(shape, lambda i: (0, 0))
    return pl.pallas_call(
        _fused_body,
        grid=(_NSTEP,),
        in_specs=[
            pl.BlockSpec((_RBLK, _ND), lambda i: (i, 0)),
            full((_ND, _HID)),
            full((1, _HID)),
            full((_HID, _HID)),
            full((1, _HID)),
            full((_HID, _HID)),
            full((1, _HID)),
            full((_HID, _HID)),
            full((1, _HID)),
            full((_APAD, _APAD)),
            full((_B, _N)),
        ],
        out_specs=pl.BlockSpec((_B, _HID), lambda i: (0, 0)),
        out_shape=jax.ShapeDtypeStruct((_B, _HID), jnp.float32),
        scratch_shapes=[
            pltpu.VMEM((_N, _HID), jnp.float32),
            pltpu.VMEM((_N, _HID), jnp.float32),
        ],
    )(patches, W_embed.T, b_embed.reshape(1, -1), W1.T, b1.reshape(1, -1),
      W2.T, b2.reshape(1, -1), W3.T, b3.reshape(1, -1),
      jnp.asarray(A), jnp.asarray(Sp))
